# Initial kernel scaffold; baseline (speedup 1.0000x reference)
import functools

import jax
import jax.numpy as jnp
from jax import lax
from jax.experimental import pallas as pl
from jax.experimental.pallas import tpu as pltpu
from jax.experimental.pallas import tpu_sc as plsc

N = 10000
D = 128
E = 320000
C = 64

NC = 2
NS = 16
NW = NC * NS
K = 128
CHUNKS = (E + NW * K - 1) // (NW * K)
EPW = K * CHUNKS
EP = EPW * NW
RPT = N // NS


def _sc_agg_body(h_hbm, src_hbm, dst_hbm, ew_hbm, zeros_hbm, out_hbm,
                 src_v, dst_v, ew_v, rows_v, acc_sh, sem):
    c = lax.axis_index("c")
    s = lax.axis_index("s")
    wid = c * NS + s

    pltpu.sync_copy(zeros_hbm.at[pl.ds(s * RPT, RPT)],
                    acc_sh.at[pl.ds(s * RPT, RPT)])
    pltpu.sync_copy(src_hbm.at[wid], src_v)
    pltpu.sync_copy(dst_hbm.at[wid], dst_v)
    pltpu.sync_copy(ew_hbm.at[wid], ew_v)
    plsc.subcore_barrier()

    def chunk_body(g, carry):
        pltpu.async_copy(h_hbm.at[src_v.at[g]], rows_v, sem).wait()

        def row_body(j, carry2):
            w = ew_v[g, j]
            for t in range(8):
                sl = pl.ds(t * 16, 16)
                rows_v[j, sl] = rows_v[j, sl] * w
            return carry2
        lax.fori_loop(0, K, row_body, 0)

        pltpu.sync_copy(rows_v, acc_sh.at[dst_v.at[g]], add=True)
        return carry

    lax.fori_loop(0, CHUNKS, chunk_body, 0)

    plsc.subcore_barrier()
    pltpu.sync_copy(acc_sh.at[pl.ds(s * RPT, RPT)],
                    out_hbm.at[c, pl.ds(s * RPT, RPT)])


_sc_agg = pl.kernel(
    _sc_agg_body,
    out_type=jax.ShapeDtypeStruct((NC, N, D), jnp.float32),
    mesh=plsc.VectorSubcoreMesh(core_axis_name="c", subcore_axis_name="s"),
    scratch_types=[
        pltpu.VMEM((CHUNKS, K), jnp.int32),
        pltpu.VMEM((CHUNKS, K), jnp.int32),
        pltpu.VMEM((CHUNKS, K), jnp.float32),
        pltpu.VMEM((K, D), jnp.float32),
        pltpu.VMEM_SHARED((N, D), jnp.float32),
        pltpu.SemaphoreType.DMA,
    ],
)


BN = 1000


def _layer_tc_body(a0, a1, h, wrel, wroot, brel, o):
    agg = a0[...] + a1[...]
    o[...] = jnp.maximum(
        jnp.dot(agg, wrel[...], preferred_element_type=jnp.float32)
        + jnp.dot(h[...], wroot[...], preferred_element_type=jnp.float32)
        + brel[...],
        0.0,
    )


def _layer_tc(a0, a1, h, wrel, wroot, brel):
    return pl.pallas_call(
        _layer_tc_body,
        grid=(N // BN,),
        in_specs=[
            pl.BlockSpec((BN, D), lambda i: (i, 0)),
            pl.BlockSpec((BN, D), lambda i: (i, 0)),
            pl.BlockSpec((BN, D), lambda i: (i, 0)),
            pl.BlockSpec((D, D), lambda i: (0, 0)),
            pl.BlockSpec((D, D), lambda i: (0, 0)),
            pl.BlockSpec((1, D), lambda i: (0, 0)),
        ],
        out_specs=pl.BlockSpec((BN, D), lambda i: (i, 0)),
        out_shape=jax.ShapeDtypeStruct((N, D), jnp.float32),
    )(a0, a1, h, wrel, wroot, brel)


def _head_tc_body(h1, h2, h3, w1, w2, w3, b, o):
    z = (jnp.dot(h1[...], w1[...], preferred_element_type=jnp.float32)
         + jnp.dot(h2[...], w2[...], preferred_element_type=jnp.float32)
         + jnp.dot(h3[...], w3[...], preferred_element_type=jnp.float32)
         + b[...])
    m = jnp.max(z, axis=-1, keepdims=True)
    ez = jnp.exp(z - m)
    lse = jnp.log(jnp.sum(ez, axis=-1, keepdims=True))
    o[...] = z - m - lse


def _head_tc(h1, h2, h3, w1, w2, w3, b):
    return pl.pallas_call(
        _head_tc_body,
        grid=(N // BN,),
        in_specs=[
            pl.BlockSpec((BN, D), lambda i: (i, 0)),
            pl.BlockSpec((BN, D), lambda i: (i, 0)),
            pl.BlockSpec((BN, D), lambda i: (i, 0)),
            pl.BlockSpec((D, C), lambda i: (0, 0)),
            pl.BlockSpec((D, C), lambda i: (0, 0)),
            pl.BlockSpec((D, C), lambda i: (0, 0)),
            pl.BlockSpec((1, C), lambda i: (0, 0)),
        ],
        out_specs=pl.BlockSpec((BN, C), lambda i: (i, 0)),
        out_shape=jax.ShapeDtypeStruct((N, C), jnp.float32),
    )(h1, h2, h3, w1, w2, w3, b)


def kernel(x, edge_index, edge_weight, W_rel_0, b_rel_0, W_root_0,
           W_rel_1, b_rel_1, W_root_1, W_rel_2, b_rel_2, W_root_2,
           W_lin, b_lin):
    pad = EP - E
    src = jnp.pad(edge_index[0], (0, pad)).reshape(NW, CHUNKS, K)
    dst = jnp.pad(edge_index[1], (0, pad)).reshape(NW, CHUNKS, K)
    ew = jnp.pad(edge_weight, (0, pad)).reshape(NW, CHUNKS, K)
    zeros = jnp.zeros((N, D), jnp.float32)

    params = [(W_rel_0, b_rel_0, W_root_0),
              (W_rel_1, b_rel_1, W_root_1),
              (W_rel_2, b_rel_2, W_root_2)]
    h = x
    hs = []
    for (wrel, brel, wroot) in params:
        parts = _sc_agg(h, src, dst, ew, zeros)
        h = _layer_tc(parts[0], parts[1], h, wrel, wroot,
                      brel.reshape(1, D))
        hs.append(h)

    return _head_tc(hs[0], hs[1], hs[2],
                    W_lin[0:D], W_lin[D:2 * D], W_lin[2 * D:3 * D],
                    b_lin.reshape(1, C))


# speedup vs baseline: 3.8763x; 3.8763x over previous
import functools

import jax
import jax.numpy as jnp
from jax import lax
from jax.experimental import pallas as pl
from jax.experimental.pallas import tpu as pltpu
from jax.experimental.pallas import tpu_sc as plsc

N = 10000
D = 128
E = 320000
C = 64

NC = 2
NS = 16
NW = NC * NS
K = 128
CHUNKS = (E + NW * K - 1) // (NW * K)
EPW = K * CHUNKS
EP = EPW * NW
N_PAD = 10240
RPT = N_PAD // NS


def _sc_agg_body(h_hbm, src_hbm, dst_hbm, ew_hbm, zeros_hbm, out_hbm,
                 src_v, dst_v, ew_v, rows_v, acc_sh, sem):
    c = lax.axis_index("c")
    s = lax.axis_index("s")
    wid = c * NS + s

    pltpu.sync_copy(zeros_hbm.at[pl.ds(s * RPT, RPT)],
                    acc_sh.at[pl.ds(s * RPT, RPT)])
    pltpu.sync_copy(src_hbm.at[wid], src_v)
    pltpu.sync_copy(dst_hbm.at[wid], dst_v)
    pltpu.sync_copy(ew_hbm.at[wid], ew_v)
    plsc.subcore_barrier()

    def chunk_body(g, carry):
        pltpu.async_copy(h_hbm.at[src_v.at[g]], rows_v, sem).wait()

        def row_body(q, carry2):
            w16 = ew_v[g, pl.ds(q * 16, 16)]
            for jj in range(16):
                w = w16[jj]
                j = q * 16 + jj
                for t in range(8):
                    sl = pl.ds(t * 16, 16)
                    rows_v[j, sl] = rows_v[j, sl] * w
            return carry2
        lax.fori_loop(0, K // 16, row_body, 0)

        pltpu.sync_copy(rows_v, acc_sh.at[dst_v.at[g]], add=True)
        return carry

    lax.fori_loop(0, CHUNKS, chunk_body, 0)

    plsc.subcore_barrier()
    pltpu.sync_copy(acc_sh.at[pl.ds(s * RPT, RPT)],
                    out_hbm.at[c, pl.ds(s * RPT, RPT)])


@functools.cache
def _make_sc_agg():
    return pl.kernel(
        _sc_agg_body,
        out_type=jax.ShapeDtypeStruct((NC, N_PAD, D), jnp.float32),
        mesh=plsc.VectorSubcoreMesh(core_axis_name="c", subcore_axis_name="s",
                                    num_cores=NC, num_subcores=NS),
        scratch_types=[
            pltpu.VMEM((CHUNKS, K), jnp.int32),
            pltpu.VMEM((CHUNKS, K), jnp.int32),
            pltpu.VMEM((CHUNKS, K), jnp.float32),
            pltpu.VMEM((K, D), jnp.float32),
            pltpu.VMEM_SHARED((N_PAD, D), jnp.float32),
            pltpu.SemaphoreType.DMA,
        ],
    )


BN = 1000


def _layer_tc_body(a0, a1, h, wrel, wroot, brel, o):
    agg = a0[...] + a1[...]
    o[...] = jnp.maximum(
        jnp.dot(agg, wrel[...], preferred_element_type=jnp.float32)
        + jnp.dot(h[...], wroot[...], preferred_element_type=jnp.float32)
        + brel[...],
        0.0,
    )


def _layer_tc(a0, a1, h, wrel, wroot, brel):
    return pl.pallas_call(
        _layer_tc_body,
        grid=(N // BN,),
        in_specs=[
            pl.BlockSpec((BN, D), lambda i: (i, 0)),
            pl.BlockSpec((BN, D), lambda i: (i, 0)),
            pl.BlockSpec((BN, D), lambda i: (i, 0)),
            pl.BlockSpec((D, D), lambda i: (0, 0)),
            pl.BlockSpec((D, D), lambda i: (0, 0)),
            pl.BlockSpec((1, D), lambda i: (0, 0)),
        ],
        out_specs=pl.BlockSpec((BN, D), lambda i: (i, 0)),
        out_shape=jax.ShapeDtypeStruct((N, D), jnp.float32),
    )(a0, a1, h, wrel, wroot, brel)


def _head_tc_body(h1, h2, h3, w1, w2, w3, b, o):
    z = (jnp.dot(h1[...], w1[...], preferred_element_type=jnp.float32)
         + jnp.dot(h2[...], w2[...], preferred_element_type=jnp.float32)
         + jnp.dot(h3[...], w3[...], preferred_element_type=jnp.float32)
         + b[...])
    m = jnp.max(z, axis=-1, keepdims=True)
    ez = jnp.exp(z - m)
    lse = jnp.log(jnp.sum(ez, axis=-1, keepdims=True))
    o[...] = z - m - lse


def _head_tc(h1, h2, h3, w1, w2, w3, b):
    return pl.pallas_call(
        _head_tc_body,
        grid=(N // BN,),
        in_specs=[
            pl.BlockSpec((BN, D), lambda i: (i, 0)),
            pl.BlockSpec((BN, D), lambda i: (i, 0)),
            pl.BlockSpec((BN, D), lambda i: (i, 0)),
            pl.BlockSpec((D, C), lambda i: (0, 0)),
            pl.BlockSpec((D, C), lambda i: (0, 0)),
            pl.BlockSpec((D, C), lambda i: (0, 0)),
            pl.BlockSpec((1, C), lambda i: (0, 0)),
        ],
        out_specs=pl.BlockSpec((BN, C), lambda i: (i, 0)),
        out_shape=jax.ShapeDtypeStruct((N, C), jnp.float32),
    )(h1, h2, h3, w1, w2, w3, b)


def kernel(x, edge_index, edge_weight, W_rel_0, b_rel_0, W_root_0,
           W_rel_1, b_rel_1, W_root_1, W_rel_2, b_rel_2, W_root_2,
           W_lin, b_lin):
    pad = EP - E
    src = jnp.pad(edge_index[0], (0, pad)).reshape(NW, CHUNKS, K)
    dst = jnp.pad(edge_index[1], (0, pad)).reshape(NW, CHUNKS, K)
    ew = jnp.pad(edge_weight, (0, pad)).reshape(NW, CHUNKS, K)
    zeros = jnp.zeros((N_PAD, D), jnp.float32)

    params = [(W_rel_0, b_rel_0, W_root_0),
              (W_rel_1, b_rel_1, W_root_1),
              (W_rel_2, b_rel_2, W_root_2)]
    h = x
    hs = []
    for (wrel, brel, wroot) in params:
        parts = _make_sc_agg()(h, src, dst, ew, zeros)
        parts = parts[:, :N]
        h = _layer_tc(parts[0], parts[1], h, wrel, wroot,
                      brel.reshape(1, D))
        hs.append(h)

    return _head_tc(hs[0], hs[1], hs[2],
                    W_lin[0:D], W_lin[D:2 * D], W_lin[2 * D:3 * D],
                    b_lin.reshape(1, C))
